# Initial kernel scaffold; baseline (speedup 1.0000x reference)
import jax
import jax.numpy as jnp
from jax.experimental import pallas as pl


def kernel(x, y, table):
    raise NotImplementedError("write your pallas kernel here")


# speedup vs baseline: 1.4708x; 1.4708x over previous
import functools

import jax
import jax.numpy as jnp
from jax import lax
from jax.experimental import pallas as pl
from jax.experimental.pallas import tpu as pltpu
from jax.experimental.pallas import tpu_sc as plsc

NC = 2
NS = 16
NW = NC * NS
CH = 32


def _lse_body(t_ref, o_ref):
    t = t_ref[...]
    m = jnp.max(t, axis=1)
    s = jnp.sum(jnp.exp(t - m[:, None]), axis=1)
    o_ref[...] = (m + jnp.log(s))[None, :]


def _finalize_body(p_ref, o_ref, *, n):
    o_ref[...] = (jnp.sum(p_ref[...]) * (1.0 / n)).reshape(1, 1)


def kernel(x, y, table):
    B, L = x.shape
    V = table.shape[0]
    N = B * L
    per_w = N // NW
    n_ch = per_w // CH

    x3 = x.reshape(NW, n_ch, CH)
    y3 = y.reshape(NW, n_ch, CH)

    lse2d = pl.pallas_call(
        _lse_body,
        out_shape=jax.ShapeDtypeStruct((1, V), jnp.float32),
    )(table)
    lse = lse2d.reshape(V)

    mesh = plsc.VectorSubcoreMesh(core_axis_name="c", subcore_axis_name="s")

    @functools.partial(
        pl.kernel,
        mesh=mesh,
        out_type=[
            jax.ShapeDtypeStruct((N, V), jnp.float32),
            jax.ShapeDtypeStruct((NW, 16), jnp.float32),
        ],
        scratch_types=[
            pltpu.VMEM((n_ch, CH), jnp.int32),
            pltpu.VMEM((n_ch, CH), jnp.int32),
            pltpu.VMEM((CH, V), jnp.float32),
            pltpu.VMEM((CH, V), jnp.float32),
            pltpu.VMEM((V,), jnp.float32),
            pltpu.VMEM((16,), jnp.float32),
            pltpu.SemaphoreType.DMA,
            pltpu.SemaphoreType.DMA,
        ],
        compiler_params=pltpu.CompilerParams(
            use_tc_tiling_on_sc=False, needs_layout_passes=False),
    )
    def _gather(x_hbm, y_hbm, table_hbm, lse_hbm, out_hbm, part_hbm,
                xi_v, yi_v, rows0, rows1, lse_v, acc_v, sem0, sem1):
        wid = lax.axis_index("s") * NC + lax.axis_index("c")
        pltpu.sync_copy(x_hbm.at[wid], xi_v)
        pltpu.sync_copy(y_hbm.at[wid], yi_v)
        pltpu.sync_copy(lse_hbm, lse_v)

        bufs = (rows0, rows1)
        sems = (sem0, sem1)
        base = wid * per_w
        rid0 = lax.iota(jnp.int32, 16)

        pltpu.async_copy(table_hbm.at[xi_v.at[0]], bufs[0], sems[0])

        def pair_body(p, acc):
            for b in range(2):
                c = 2 * p + b
                pltpu.make_async_copy(
                    table_hbm.at[xi_v.at[c]], bufs[b], sems[b]).wait()

                @pl.when(c + 1 < n_ch)
                def _():
                    pltpu.async_copy(
                        table_hbm.at[xi_v.at[c + 1]], bufs[1 - b],
                        sems[1 - b])

                for k in range(CH // 16):
                    xv = xi_v[c, pl.ds(k * 16, 16)]
                    yv = yi_v[c, pl.ds(k * 16, 16)]
                    picked = plsc.load_gather(bufs[b], [rid0 + (k * 16), yv])
                    lse_g = plsc.load_gather(lse_v, [xv])
                    acc = acc + (lse_g - picked)

                pltpu.sync_copy(bufs[b], out_hbm.at[pl.ds(base + c * CH, CH)])
            return acc

        acc = lax.fori_loop(0, n_ch // 2, pair_body,
                            jnp.zeros((16,), jnp.float32))
        acc_v[...] = acc
        pltpu.sync_copy(acc_v, part_hbm.at[wid])

    logits_flat, partials = _gather(x3, y3, table, lse)

    loss2d = pl.pallas_call(
        functools.partial(_finalize_body, n=N),
        out_shape=jax.ShapeDtypeStruct((1, 1), jnp.float32),
    )(partials)
    loss = loss2d[0, 0]

    return (logits_flat, loss)
